# baseline (speedup 1.0000x reference)
import functools

import jax
import jax.numpy as jnp
from jax import lax
from jax.experimental import pallas as pl
from jax.experimental.pallas import tpu as pltpu
from jax.experimental.pallas import tpu_sc as plsc

_B = 16384
_NW = 32
_BPW = _B // _NW
_GCH = 128
_NCH = _BPW // _GCH
_N = 1000000
_NPAD = 1000448
_CH = _NPAD // _NW
_UN = 16
_OUTER = 122
_REM = _CH // 16 - _OUTER * _UN
_NACC = 4


def _sum_body(xk, t_hbm, px_hbm, tgv_hbm, ik, tg, sb, pp, sem_t, sem_k):
    cid = lax.axis_index("c")
    sid = lax.axis_index("s")
    wid = sid * 2 + cid
    cp_t = pltpu.async_copy(t_hbm.at[pl.ds(wid * _CH, _CH)], sb, sem_t)
    pltpu.sync_copy(xk.at[wid], ik)
    cps = []
    for c in range(_NCH):
        d = pl.ds(c * _GCH, _GCH)
        cps.append(pltpu.async_copy(t_hbm.at[ik.at[c]], tg.at[d], sem_k))
    cp_t.wait()

    def sbody(t, accs):
        base = t * (_UN * 16)
        accs = list(accs)
        for k in range(_UN):
            v = sb[pl.ds(base + k * 16, 16)]
            accs[k % _NACC] = accs[k % _NACC] + v
        return tuple(accs)

    zf = jnp.zeros((16,), jnp.float32)
    accs = lax.fori_loop(0, _OUTER, sbody, (zf,) * _NACC)
    acc = accs[0]
    for a in accs[1:]:
        acc = acc + a
    for k in range(_REM):
        acc = acc + sb[pl.ds((_OUTER * _UN + k) * 16, 16)]
    pp[...] = acc
    pltpu.sync_copy(pp, px_hbm.at[wid])
    for cp in cps:
        cp.wait()
    pltpu.sync_copy(tg, tgv_hbm.at[pl.ds(wid * _BPW, _BPW)])


def _gather_body(xi, xj, r_hbm, px_hbm, tgv_hbm, o_hbm,
                 ii, ij, rg, jg, tg, po, shv,
                 sem_i, sem_j, sem_k, sem_p):
    cid = lax.axis_index("c")
    sid = lax.axis_index("s")
    wid = sid * 2 + cid
    cp_p = pltpu.async_copy(px_hbm, shv, sem_p)
    cp_g = pltpu.async_copy(tgv_hbm.at[pl.ds(wid * _BPW, _BPW)], tg, sem_k)
    pltpu.sync_copy(xi.at[wid], ii)
    pltpu.sync_copy(xj.at[wid], ij)
    cps = [cp_g]
    for c in range(_NCH):
        d = pl.ds(c * _GCH, _GCH)
        cps.append(pltpu.async_copy(r_hbm.at[ii.at[c]], rg.at[d], sem_i))
        cps.append(pltpu.async_copy(r_hbm.at[ij.at[c]], jg.at[d], sem_j))
    cp_p.wait()
    v = shv[0]
    for r in range(1, _NW):
        v = v + shv[r]
    tot = v[0]
    for l in range(1, 16):
        tot = tot + v[l]
    inv = jnp.float32(1.0) / jnp.full((16,), tot, jnp.float32)

    for cp in cps:
        cp.wait()
    for t in range(_BPW // 16):
        sl = pl.ds(t * 16, 16)
        po[sl] = tg[sl] * inv * (rg[sl] - jg[sl])
    pltpu.sync_copy(po, o_hbm.at[pl.ds(wid * _BPW, _BPW)])


@jax.jit
def kernel(x, R, Theta):
    xt = x.T
    xi = xt[0].reshape(_NW, _NCH, _GCH)
    xj = xt[1].reshape(_NW, _NCH, _GCH)
    xk = xt[2].reshape(_NW, _NCH, _GCH)
    tflat = jnp.pad(Theta.T, ((0, 0), (0, _NPAD - _N))).reshape(_NPAD)
    rflat = jnp.pad(R.T, ((0, 0), (0, _NPAD - _N))).reshape(_NPAD)
    mesh = plsc.VectorSubcoreMesh(core_axis_name="c", subcore_axis_name="s")

    px, tgv = functools.partial(
        pl.kernel,
        mesh=mesh,
        out_type=(jax.ShapeDtypeStruct((_NW, 16), jnp.float32),
                  jax.ShapeDtypeStruct((_B,), jnp.float32)),
        scratch_types=[
            pltpu.VMEM((_NCH, _GCH), jnp.int32),
            pltpu.VMEM((_BPW,), jnp.float32),
            pltpu.VMEM((_CH,), jnp.float32),
            pltpu.VMEM((16,), jnp.float32),
            pltpu.SemaphoreType.DMA,
            pltpu.SemaphoreType.DMA,
        ],
    )(_sum_body)(xk, tflat)

    sc = functools.partial(
        pl.kernel,
        mesh=mesh,
        out_type=jax.ShapeDtypeStruct((_B,), jnp.float32),
        scratch_types=[
            pltpu.VMEM((_NCH, _GCH), jnp.int32),
            pltpu.VMEM((_NCH, _GCH), jnp.int32),
            pltpu.VMEM((_BPW,), jnp.float32),
            pltpu.VMEM((_BPW,), jnp.float32),
            pltpu.VMEM((_BPW,), jnp.float32),
            pltpu.VMEM((_BPW,), jnp.float32),
            pltpu.VMEM((_NW, 16), jnp.float32),
            pltpu.SemaphoreType.DMA,
            pltpu.SemaphoreType.DMA,
            pltpu.SemaphoreType.DMA,
            pltpu.SemaphoreType.DMA,
        ],
    )(_gather_body)
    p = sc(xi, xj, rflat, px, tgv)
    return p.reshape(_B, 1)


# speedup vs baseline: 1.0106x; 1.0106x over previous
import functools

import jax
import jax.numpy as jnp
from jax import lax
from jax.experimental import pallas as pl
from jax.experimental.pallas import tpu as pltpu
from jax.experimental.pallas import tpu_sc as plsc

_B = 16384
_NW = 32
_BPW = _B // _NW
_GCH = 128
_NCH = _BPW // _GCH
_N = 1000000
_NPAD = 1000448
_CH = _NPAD // _NW
_UN = 16
_OUTER = 122
_REM = _CH // 16 - _OUTER * _UN
_NACC = 4


def _sum_body(t_hbm, px_hbm, sb, pp, sem_t):
    cid = lax.axis_index("c")
    sid = lax.axis_index("s")
    wid = sid * 2 + cid
    pltpu.async_copy(t_hbm.at[pl.ds(wid * _CH, _CH)], sb, sem_t).wait()

    def sbody(t, accs):
        base = t * (_UN * 16)
        accs = list(accs)
        for k in range(_UN):
            v = sb[pl.ds(base + k * 16, 16)]
            accs[k % _NACC] = accs[k % _NACC] + v
        return tuple(accs)

    zf = jnp.zeros((16,), jnp.float32)
    accs = lax.fori_loop(0, _OUTER, sbody, (zf,) * _NACC)
    acc = accs[0]
    for a in accs[1:]:
        acc = acc + a
    for k in range(_REM):
        acc = acc + sb[pl.ds((_OUTER * _UN + k) * 16, 16)]
    pp[...] = acc
    pltpu.sync_copy(pp, px_hbm.at[wid])


def _gather_body(xi, xj, xk, r_hbm, t_hbm, px_hbm, o_hbm,
                 ii, ij, ik, rg, jg, tg, po, shv,
                 sem_i, sem_j, sem_k, sem_p):
    cid = lax.axis_index("c")
    sid = lax.axis_index("s")
    wid = sid * 2 + cid
    cp_p = pltpu.async_copy(px_hbm, shv, sem_p)
    pltpu.sync_copy(xi.at[wid], ii)
    pltpu.sync_copy(xj.at[wid], ij)
    pltpu.sync_copy(xk.at[wid], ik)
    cps = []
    for c in range(_NCH):
        d = pl.ds(c * _GCH, _GCH)
        cps.append(pltpu.async_copy(r_hbm.at[ii.at[c]], rg.at[d], sem_i))
        cps.append(pltpu.async_copy(r_hbm.at[ij.at[c]], jg.at[d], sem_j))
        cps.append(pltpu.async_copy(t_hbm.at[ik.at[c]], tg.at[d], sem_k))
    cp_p.wait()
    v = shv[0]
    for r in range(1, _NW):
        v = v + shv[r]
    tot = v[0]
    for l in range(1, 16):
        tot = tot + v[l]
    inv = jnp.float32(1.0) / jnp.full((16,), tot, jnp.float32)

    for cp in cps:
        cp.wait()
    for t in range(_BPW // 16):
        sl = pl.ds(t * 16, 16)
        po[sl] = tg[sl] * inv * (rg[sl] - jg[sl])
    pltpu.sync_copy(po, o_hbm.at[pl.ds(wid * _BPW, _BPW)])


@jax.jit
def kernel(x, R, Theta):
    xt = x.T
    xi = xt[0].reshape(_NW, _NCH, _GCH)
    xj = xt[1].reshape(_NW, _NCH, _GCH)
    xk = xt[2].reshape(_NW, _NCH, _GCH)
    tflat = jnp.pad(Theta.T, ((0, 0), (0, _NPAD - _N))).reshape(_NPAD)
    rflat = jnp.pad(R.T, ((0, 0), (0, _NPAD - _N))).reshape(_NPAD)
    mesh = plsc.VectorSubcoreMesh(core_axis_name="c", subcore_axis_name="s")

    px = functools.partial(
        pl.kernel,
        mesh=mesh,
        out_type=jax.ShapeDtypeStruct((_NW, 16), jnp.float32),
        scratch_types=[
            pltpu.VMEM((_CH,), jnp.float32),
            pltpu.VMEM((16,), jnp.float32),
            pltpu.SemaphoreType.DMA,
        ],
    )(_sum_body)(tflat)

    sc = functools.partial(
        pl.kernel,
        mesh=mesh,
        out_type=jax.ShapeDtypeStruct((_B,), jnp.float32),
        scratch_types=[
            pltpu.VMEM((_NCH, _GCH), jnp.int32),
            pltpu.VMEM((_NCH, _GCH), jnp.int32),
            pltpu.VMEM((_NCH, _GCH), jnp.int32),
            pltpu.VMEM((_BPW,), jnp.float32),
            pltpu.VMEM((_BPW,), jnp.float32),
            pltpu.VMEM((_BPW,), jnp.float32),
            pltpu.VMEM((_BPW,), jnp.float32),
            pltpu.VMEM((_NW, 16), jnp.float32),
            pltpu.SemaphoreType.DMA,
            pltpu.SemaphoreType.DMA,
            pltpu.SemaphoreType.DMA,
            pltpu.SemaphoreType.DMA,
        ],
    )(_gather_body)
    p = sc(xi, xj, xk, rflat, tflat, px)
    return p.reshape(_B, 1)
